# Initial kernel scaffold; baseline (speedup 1.0000x reference)
import jax
import jax.numpy as jnp
from jax.experimental import pallas as pl


def kernel(x, params):
    raise NotImplementedError("write your pallas kernel here")


# speedup vs baseline: 2.1347x; 2.1347x over previous
import functools

import jax
import jax.numpy as jnp
import numpy as np
from jax import lax
from jax.experimental import pallas as pl
from jax.experimental.pallas import tpu as pltpu
from jax.experimental.pallas import tpu_sc as plsc

_B = 4
_NPTS = 10000
_NIDX = 1000
_NCH = 20
_ROW = 11000
_NJOBS = _B * _NCH
_NW = 32
_PAD = 1024


def _gather_body(xflat_hbm, out_hbm, fidx_v, idx_v, rows_v, sem):
    wid = lax.axis_index("s") * 2 + lax.axis_index("c")

    def run_job(j):
        b = j // _NCH
        foff = b * (_NCH * _ROW) + _NPTS
        pltpu.sync_copy(xflat_hbm.at[pl.ds(foff, _NIDX)],
                        fidx_v.at[pl.ds(0, _NIDX)])
        fidx_v[pl.ds(_NIDX, 16)] = jnp.zeros((16,), jnp.float32)
        fidx_v[pl.ds(_PAD - 16, 16)] = jnp.zeros((16,), jnp.float32)
        base = j * _ROW
        for t in range(_PAD // 16):
            chunk = fidx_v[pl.ds(t * 16, 16)]
            idx_v[pl.ds(t * 16, 16)] = chunk.astype(jnp.int32) + base
        copies = []
        for k in range(_PAD // 128):
            sl = pl.ds(k * 128, 128)
            copies.append(
                pltpu.async_copy(xflat_hbm.at[idx_v.at[sl]], rows_v.at[sl],
                                 sem))
        for cp in copies:
            cp.wait()
        pltpu.sync_copy(rows_v.at[pl.ds(0, _NIDX)], out_hbm.at[j])

    run_job(wid)
    run_job(wid + _NW)

    @pl.when(wid + 2 * _NW < _NJOBS)
    def _():
        run_job(wid + 2 * _NW)


def _sc_gather(x):
    xflat = x.reshape(-1)
    mesh = plsc.VectorSubcoreMesh(core_axis_name="c", subcore_axis_name="s")
    k = pl.kernel(
        _gather_body,
        out_type=jax.ShapeDtypeStruct((_NJOBS, _NIDX), jnp.float32),
        mesh=mesh,
        scratch_types=[
            pltpu.VMEM((_PAD,), jnp.float32),
            pltpu.VMEM((_PAD,), jnp.int32),
            pltpu.VMEM((_PAD,), jnp.float32),
            pltpu.SemaphoreType.DMA,
        ],
        compiler_params=pltpu.CompilerParams(use_tc_tiling_on_sc=False),
    )
    return k(xflat)


_BN_S = np.float32(1.0 / np.sqrt(1.0 + 1e-5))


def _dense_body(a_ref, w1_ref, bn1s_ref, bn1b_ref, w2_ref, w3_ref, w4_ref,
                w5_ref, m1_ref, m2_ref, m3_ref, bns_ref, bnb_ref, o_ref):
    rows = []
    bn1s = bn1s_ref[...]
    bn1b = bn1b_ref[...]
    for b in range(_B):
        z = jax.lax.dot(w1_ref[...], a_ref[b],
                        precision=jax.lax.Precision.HIGHEST,
                        preferred_element_type=jnp.float32)
        z = jnp.maximum(z * (_BN_S * bn1s) + bn1b, 0.0)
        rows.append(jnp.max(z, axis=1).reshape(1, 64))
    x1 = jnp.concatenate(rows, axis=0)

    bns = bns_ref[...]
    bnb = bnb_ref[...]

    def step(xv, wt_ref, off, width):
        z = jax.lax.dot(xv, wt_ref[...],
                        precision=jax.lax.Precision.HIGHEST,
                        preferred_element_type=jnp.float32)
        s = bns[:, off:off + width]
        t = bnb[:, off:off + width]
        return jnp.maximum(z * (_BN_S * s) + t, 0.0)

    x2 = step(x1, w2_ref, 0, 64)
    x3 = step(x2, w3_ref, 64, 128)
    x4 = step(x3, w4_ref, 192, 256)
    cat = jnp.concatenate([x1, x2, x3, x4], axis=1)
    h5 = step(cat, w5_ref, 448, 1024)
    h6 = step(h5, m1_ref, 1472, 512)
    h7 = step(h6, m2_ref, 1984, 256)
    o_ref[...] = jax.lax.dot(h7, m3_ref[...],
                             precision=jax.lax.Precision.HIGHEST,
                             preferred_element_type=jnp.float32)


def _dense_chain(a, p):
    w2t = p['conv2_w'][:, 64:].T
    w3t = p['conv3_w'][:, 64:].T
    w4t = p['conv4_w'][:, 128:].T
    w5t = p['conv5_w'].T
    m1t = p['mlp1_w'].T
    m2t = p['mlp2_w'].T
    m3t = p['mlp3_w'].T
    bns = jnp.concatenate([p['bn2_w'], p['bn3_w'], p['bn4_w'], p['bn5_w'],
                           p['bn6_w'], p['bn7_w']]).reshape(1, -1)
    bnb = jnp.concatenate([p['bn2_b'], p['bn3_b'], p['bn4_b'], p['bn5_b'],
                           p['bn6_b'], p['bn7_b']]).reshape(1, -1)
    return pl.pallas_call(
        _dense_body,
        out_shape=jax.ShapeDtypeStruct((_B, 40), jnp.float32),
    )(a, p['conv1_w'], p['bn1_w'].reshape(-1, 1), p['bn1_b'].reshape(-1, 1),
      w2t, w3t, w4t, w5t, m1t, m2t, m3t, bns, bnb)


@jax.jit
def kernel(x, params):
    g = _sc_gather(x)
    a = g.reshape(_B, _NIDX, _NCH)
    return _dense_chain(a, params)
